# baseline (speedup 1.0000x reference)
import functools

import jax
import jax.numpy as jnp
from jax.experimental import pallas as pl
from jax.experimental.pallas import tpu as pltpu

N, D, H, E = 2048, 1024, 2048, 8
TILE = 256
NT = N // TILE


def _router_tile(x_tile, gate):
    logits = jnp.dot(x_tile, gate, preferred_element_type=jnp.float32)
    idx8 = jax.lax.broadcasted_iota(jnp.int32, (TILE, E), 1)
    m1 = jnp.max(logits, axis=1, keepdims=True)
    e1 = jnp.min(jnp.where(logits == m1, idx8, E), axis=1, keepdims=True)
    oh1 = idx8 == e1
    l2 = jnp.where(oh1, -jnp.inf, logits)
    m2 = jnp.max(l2, axis=1, keepdims=True)
    e2 = jnp.min(jnp.where(l2 == m2, idx8, E), axis=1, keepdims=True)
    oh2 = idx8 == e2
    u = jnp.exp(m2 - m1)
    g1 = 1.0 / (1.0 + u)
    g2 = u / (1.0 + u)
    return oh1 * g1 + oh2 * g2


def _moe_body(x_ref, gate_ref, w1_ref, b1_ref, w2_ref, b2_ref, out_ref, gt_ref):
    e = pl.program_id(1)

    @pl.when(e == 0)
    def _():
        gt_ref[...] = _router_tile(x_ref[...], gate_ref[...])

    onehot = jax.lax.broadcasted_iota(jnp.int32, (TILE, E), 1) == e
    w = jnp.sum(jnp.where(onehot, gt_ref[...], 0.0), axis=1, keepdims=True)
    h = jnp.maximum(
        jnp.dot(x_ref[...], w1_ref[0], preferred_element_type=jnp.float32)
        + b1_ref[0],
        0.0,
    )
    y = jnp.dot(h, w2_ref[0], preferred_element_type=jnp.float32) + b2_ref[0]

    @pl.when(e == 0)
    def _():
        out_ref[...] = w * y

    @pl.when(e != 0)
    def _():
        out_ref[...] += w * y


@functools.partial(jax.jit, static_argnames=())
def _moe(x2d, gate, w1, b1, w2, b2):
    return pl.pallas_call(
        _moe_body,
        grid=(NT, E),
        in_specs=[
            pl.BlockSpec((TILE, D), lambda t, e: (t, 0)),
            pl.BlockSpec((D, E), lambda t, e: (0, 0)),
            pl.BlockSpec((1, D, H), lambda t, e: (e, 0, 0)),
            pl.BlockSpec((1, 1, H), lambda t, e: (e, 0, 0)),
            pl.BlockSpec((1, H, D), lambda t, e: (e, 0, 0)),
            pl.BlockSpec((1, 1, D), lambda t, e: (e, 0, 0)),
        ],
        out_specs=pl.BlockSpec((TILE, D), lambda t, e: (t, 0)),
        out_shape=jax.ShapeDtypeStruct((N, D), jnp.float32),
        scratch_shapes=[pltpu.VMEM((TILE, E), jnp.float32)],
        compiler_params=pltpu.CompilerParams(
            dimension_semantics=("arbitrary", "arbitrary"),
        ),
    )(x2d, gate, w1, b1.reshape(E, 1, H), w2, b2.reshape(E, 1, D))


def kernel(x, gate, w1, b1, w2, b2):
    out = _moe(x.reshape(N, D), gate, w1, b1, w2, b2)
    return out.reshape(1, N, D)


# speedup vs baseline: 1.9790x; 1.9790x over previous
import functools

import jax
import jax.numpy as jnp
from jax import lax
from jax.experimental import pallas as pl
from jax.experimental.pallas import tpu as pltpu
from jax.experimental.pallas import tpu_sc as plsc

N, D, H, E = 2048, 1024, 2048, 8
TILE = 256
NT = N // TILE
MAX_TILES = 24
MAX_ROWS = MAX_TILES * TILE
EMAP_PAD = 32

_HI = jax.lax.Precision.HIGHEST


def _router_body(x_ref, gate_ref, r1_ref, r2_ref, g1_ref, g2_ref, emap_ref,
                 m_ref, incl_ref):
    logits = jnp.dot(x_ref[...], gate_ref[...],
                     preferred_element_type=jnp.float32)
    idx8 = lax.broadcasted_iota(jnp.int32, (N, E), 1)
    m1 = jnp.max(logits, axis=1, keepdims=True)
    e1 = jnp.min(jnp.where(logits == m1, idx8, E), axis=1, keepdims=True)
    oh1 = idx8 == e1
    l2 = jnp.where(oh1, -jnp.inf, logits)
    m2 = jnp.max(l2, axis=1, keepdims=True)
    e2 = jnp.min(jnp.where(l2 == m2, idx8, E), axis=1, keepdims=True)
    oh2 = idx8 == e2
    u = jnp.exp(m2 - m1)
    g1_ref[...] = 1.0 / (1.0 + u)
    g2_ref[...] = u / (1.0 + u)

    mask = (oh1 | oh2).astype(jnp.float32)
    m_ref[...] = mask
    rr = lax.broadcasted_iota(jnp.int32, (TILE, TILE), 0)
    cc = lax.broadcasted_iota(jnp.int32, (TILE, TILE), 1)
    ltri = (rr >= cc).astype(jnp.float32)

    def body(i, base):
        chunk = m_ref[pl.ds(i * TILE, TILE), :]
        incl = jnp.dot(ltri, chunk, preferred_element_type=jnp.float32,
                       precision=_HI) + base
        incl_ref[pl.ds(i * TILE, TILE), :] = incl
        return incl[TILE - 1:TILE, :]

    counts = lax.fori_loop(0, N // TILE, body, jnp.zeros((1, E), jnp.float32))

    pc = jnp.floor((counts + (TILE - 1)) / TILE) * TILE
    r8 = lax.broadcasted_iota(jnp.int32, (E, E), 0)
    c8 = lax.broadcasted_iota(jnp.int32, (E, E), 1)
    ustrict = (r8 < c8).astype(jnp.float32)
    base_e = jnp.dot(pc, ustrict, preferred_element_type=jnp.float32,
                     precision=_HI)

    excl = incl_ref[...] - mask
    dest = excl + base_e
    r1_ref[...] = jnp.sum(jnp.where(oh1, dest, 0.0), axis=1,
                          keepdims=True).astype(jnp.int32)
    r2_ref[...] = jnp.sum(jnp.where(oh2, dest, 0.0), axis=1,
                          keepdims=True).astype(jnp.int32)

    starts = base_e / TILE
    tt = lax.broadcasted_iota(jnp.int32, (EMAP_PAD, E), 0).astype(jnp.float32)
    ge = (tt >= starts).astype(jnp.float32)
    emap_ref[...] = (jnp.sum(ge, axis=1, keepdims=True) - 1.0).astype(jnp.int32)


def _router(x2d, gate):
    return pl.pallas_call(
        _router_body,
        in_specs=[
            pl.BlockSpec((N, D), lambda: (0, 0)),
            pl.BlockSpec((D, E), lambda: (0, 0)),
        ],
        out_specs=[
            pl.BlockSpec((N, 1), lambda: (0, 0)),
            pl.BlockSpec((N, 1), lambda: (0, 0)),
            pl.BlockSpec((N, 1), lambda: (0, 0)),
            pl.BlockSpec((N, 1), lambda: (0, 0)),
            pl.BlockSpec((EMAP_PAD, 1), lambda: (0, 0)),
        ],
        out_shape=[
            jax.ShapeDtypeStruct((N, 1), jnp.int32),
            jax.ShapeDtypeStruct((N, 1), jnp.int32),
            jax.ShapeDtypeStruct((N, 1), jnp.float32),
            jax.ShapeDtypeStruct((N, 1), jnp.float32),
            jax.ShapeDtypeStruct((EMAP_PAD, 1), jnp.int32),
        ],
        scratch_shapes=[
            pltpu.VMEM((N, E), jnp.float32),
            pltpu.VMEM((N, E), jnp.float32),
        ],
    )(x2d, gate)


_NC = 2
_NW = 32
TOK_W = N // _NW


def _sc_scatter_body(x_hbm, r1_hbm, r2_hbm, xs_hbm, i1v, i2v, xbuf, sem):
    wid = lax.axis_index("s") * _NC + lax.axis_index("c")
    base = wid * TOK_W
    pltpu.sync_copy(x_hbm.at[pl.ds(base, TOK_W)], xbuf)
    pltpu.sync_copy(r1_hbm.at[pl.ds(base, TOK_W)], i1v)
    pltpu.sync_copy(r2_hbm.at[pl.ds(base, TOK_W)], i2v)
    pltpu.async_copy(xbuf, xs_hbm.at[i1v], sem).wait()
    pltpu.async_copy(xbuf, xs_hbm.at[i2v], sem).wait()


def _sc_gather_body(y_hbm, r1_hbm, r2_hbm, z1_hbm, z2_hbm, i1v, i2v, buf, sem):
    wid = lax.axis_index("s") * _NC + lax.axis_index("c")
    base = wid * TOK_W
    pltpu.sync_copy(r1_hbm.at[pl.ds(base, TOK_W)], i1v)
    pltpu.sync_copy(r2_hbm.at[pl.ds(base, TOK_W)], i2v)
    pltpu.async_copy(y_hbm.at[i1v], buf, sem).wait()
    pltpu.sync_copy(buf, z1_hbm.at[pl.ds(base, TOK_W)])
    pltpu.async_copy(y_hbm.at[i2v], buf, sem).wait()
    pltpu.sync_copy(buf, z2_hbm.at[pl.ds(base, TOK_W)])


@functools.cache
def _sc_kernels():
    mesh = plsc.VectorSubcoreMesh(core_axis_name="c", subcore_axis_name="s")
    common_scratch = [
        pltpu.VMEM((TOK_W,), jnp.int32),
        pltpu.VMEM((TOK_W,), jnp.int32),
        pltpu.VMEM((TOK_W, D), jnp.float32),
        pltpu.SemaphoreType.DMA,
    ]
    scatter = pl.kernel(
        _sc_scatter_body,
        mesh=mesh,
        out_type=jax.ShapeDtypeStruct((MAX_ROWS, D), jnp.float32),
        scratch_types=list(common_scratch),
    )
    gather = pl.kernel(
        _sc_gather_body,
        mesh=mesh,
        out_type=[
            jax.ShapeDtypeStruct((N, D), jnp.float32),
            jax.ShapeDtypeStruct((N, D), jnp.float32),
        ],
        scratch_types=list(common_scratch),
    )
    return scatter, gather


def _gemm_body(emap_ref, xs_ref, w1_ref, b1_ref, w2_ref, b2_ref, y_ref):
    del emap_ref
    h = jnp.maximum(
        jnp.dot(xs_ref[...], w1_ref[0], preferred_element_type=jnp.float32)
        + b1_ref[0],
        0.0,
    )
    y_ref[...] = (
        jnp.dot(h, w2_ref[0], preferred_element_type=jnp.float32) + b2_ref[0]
    )


def _gemm(emap, xs, w1, b1, w2, b2):
    grid_spec = pltpu.PrefetchScalarGridSpec(
        num_scalar_prefetch=1,
        grid=(MAX_TILES,),
        in_specs=[
            pl.BlockSpec((TILE, D), lambda t, emap: (t, 0)),
            pl.BlockSpec((1, D, H), lambda t, emap: (emap[t], 0, 0)),
            pl.BlockSpec((1, 1, H), lambda t, emap: (emap[t], 0, 0)),
            pl.BlockSpec((1, H, D), lambda t, emap: (emap[t], 0, 0)),
            pl.BlockSpec((1, 1, D), lambda t, emap: (emap[t], 0, 0)),
        ],
        out_specs=pl.BlockSpec((TILE, D), lambda t, emap: (t, 0)),
    )
    return pl.pallas_call(
        _gemm_body,
        grid_spec=grid_spec,
        out_shape=jax.ShapeDtypeStruct((MAX_ROWS, D), jnp.float32),
        compiler_params=pltpu.CompilerParams(
            dimension_semantics=("arbitrary",),
        ),
    )(emap, xs, w1, b1.reshape(E, 1, H), w2, b2.reshape(E, 1, D))


def _combine_body(z1_ref, z2_ref, g1_ref, g2_ref, out_ref):
    out_ref[...] = g1_ref[...] * z1_ref[...] + g2_ref[...] * z2_ref[...]


def _combine(z1, z2, g1, g2):
    return pl.pallas_call(
        _combine_body,
        grid=(NT,),
        in_specs=[
            pl.BlockSpec((TILE, D), lambda t: (t, 0)),
            pl.BlockSpec((TILE, D), lambda t: (t, 0)),
            pl.BlockSpec((TILE, 1), lambda t: (t, 0)),
            pl.BlockSpec((TILE, 1), lambda t: (t, 0)),
        ],
        out_specs=pl.BlockSpec((TILE, D), lambda t: (t, 0)),
        out_shape=jax.ShapeDtypeStruct((N, D), jnp.float32),
    )(z1, z2, g1, g2)


def kernel(x, gate, w1, b1, w2, b2):
    x2d = x.reshape(N, D)
    r1, r2, g1, g2, emap = _router(x2d, gate)
    r1f, r2f = r1.reshape(N), r2.reshape(N)
    sc_scatter, sc_gather = _sc_kernels()
    xs = sc_scatter(x2d, r1f, r2f)
    y = _gemm(emap.reshape(EMAP_PAD), xs, w1, b1, w2, b2)
    z1, z2 = sc_gather(y, r1f, r2f)
    out = _combine(z1, z2, g1, g2)
    return out.reshape(1, N, D)
